# Initial kernel scaffold; baseline (speedup 1.0000x reference)
import jax
import jax.numpy as jnp
from jax.experimental import pallas as pl


def kernel(x, edge_index, edge_attr, batch, W1, b1, W2, b2, W3, b3, Wlin, blin):
    raise NotImplementedError("write your pallas kernel here")


# speedup vs baseline: 4.2204x; 4.2204x over previous
import functools

import jax
import jax.numpy as jnp
from jax import lax
from jax.experimental import pallas as pl
from jax.experimental.pallas import tpu as pltpu
from jax.experimental.pallas import tpu_sc as plsc

N = 100000
E = 1600000
MOL = 9
H = 64
G = 512
NC = 2
NS = 16
LANES = 16

EP = 1605632
EPAD = EP - E
EB = EP // 128
AGG_BLOCKS = 49
DEG_BLOCKS = 49

NR = 100096
ROWS_PER_TILE = 6256
ZCH = [1024] * 6 + [112]

NP = 102400
POOL_ROWS = 3200
POOL_CH = [512] * 6 + [128]
GP = G + 1

_mesh = plsc.VectorSubcoreMesh(
    core_axis_name="c", subcore_axis_name="s", num_cores=NC, num_subcores=NS)
_sc_params = pltpu.CompilerParams(
    use_tc_tiling_on_sc=False, internal_scratch_in_bytes=65536)


@functools.partial(
    pl.kernel,
    out_type=jax.ShapeDtypeStruct((2 * NR,), jnp.float32),
    mesh=_mesh,
    scratch_types=[
        pltpu.VMEM((8, 128), jnp.int32),
        pltpu.VMEM((128,), jnp.float32),
        pltpu.VMEM((1024,), jnp.float32),
        pltpu.VMEM_SHARED((NR + 8,), jnp.float32),
    ],
)
def _sc_deg(dst_hbm, out_hbm, dbuf, onesb, stage, degacc):
    c = lax.axis_index("c")
    s = lax.axis_index("s")
    wid = c * NS + s

    def initbuf(i, _):
        onesb[pl.ds(i * 16, 16)] = jnp.full((16,), 1.0, jnp.float32)
        return 0
    lax.fori_loop(0, 8, initbuf, 0)

    def zinit(i, _):
        stage[pl.ds(i * 16, 16)] = jnp.zeros((16,), jnp.float32)
        return 0
    lax.fori_loop(0, 64, zinit, 0)

    r0 = s * ROWS_PER_TILE
    off = 0
    for sz in ZCH:
        pltpu.sync_copy(stage.at[pl.ds(0, sz)], degacc.at[pl.ds(r0 + off, sz)])
        off += sz
    plsc.subcore_barrier()

    def block(bi, _):
        g = wid * DEG_BLOCKS + bi
        pltpu.sync_copy(dst_hbm.at[pl.ds(8 * g, 8)], dbuf)
        for j in range(8):
            pltpu.sync_copy(onesb, degacc.at[dbuf.at[j]], add=True)
        return 0
    lax.fori_loop(0, DEG_BLOCKS, block, 0)
    plsc.subcore_barrier()

    off = 0
    for sz in ZCH:
        pltpu.sync_copy(degacc.at[pl.ds(r0 + off, sz)], stage.at[pl.ds(0, sz)])
        pltpu.sync_copy(stage.at[pl.ds(0, sz)],
                        out_hbm.at[pl.ds(c * NR + r0 + off, sz)])
        off += sz


NH = 50048
RPT_H = 3128
ZCH_H = [1024, 1024, 1024, 56]
DUMP = NH


@functools.partial(
    pl.kernel,
    out_type=[jax.ShapeDtypeStruct((NR, 16), jnp.float32) for _ in range(4)],
    mesh=_mesh,
    scratch_types=[
        pltpu.VMEM((16, 128), jnp.int32),
        pltpu.VMEM((16, 128), jnp.int32),
        pltpu.VMEM((16, 128), jnp.int32),
        pltpu.VMEM((16, 128), jnp.int32),
        pltpu.VMEM((16, 128, 16), jnp.float32),
        pltpu.VMEM((1024, 16), jnp.float32),
        pltpu.VMEM_SHARED((NH + 8, 16), jnp.float32),
        pltpu.SemaphoreType.DMA,
        pltpu.SemaphoreType.DMA,
    ],
    compiler_params=_sc_params,
)
def _sc_aggr(src_hbm, dst_hbm, y4_hbm, o0, o1, o2, o3,
             sbuf, dbuf, gbuf, dribuf, rows, zbuf, accum, semg, sems):
    c = lax.axis_index("c")
    s = lax.axis_index("s")

    def zinit(i, _):
        zbuf[i, :] = jnp.zeros((16,), jnp.float32)
        return 0
    lax.fori_loop(0, 1024, zinit, 0)

    r0 = s * RPT_H
    outs = (o0, o1, o2, o3)

    for si in range(2):
        p = 2 * c + si
        for hf in range(2):
            lo = hf * NH

            off = 0
            for sz in ZCH_H:
                pltpu.sync_copy(zbuf.at[pl.ds(0, sz)],
                                accum.at[pl.ds(r0 + off, sz)])
                off += sz
            plsc.subcore_barrier()

            def block(bi, _):
                base_row = 16 * (s * AGG_BLOCKS + bi)
                pltpu.sync_copy(src_hbm.at[pl.ds(base_row, 16)], sbuf)
                pltpu.sync_copy(dst_hbm.at[pl.ds(base_row, 16)], dbuf)
                for j in range(16):
                    for k in range(8):
                        v = sbuf[j, pl.ds(16 * k, 16)]
                        gbuf[j, pl.ds(16 * k, 16)] = 4 * v + p
                        d = dbuf[j, pl.ds(16 * k, 16)]
                        r = d - lo
                        ok = (r >= 0) & (r < NH)
                        dribuf[j, pl.ds(16 * k, 16)] = jnp.where(ok, r, DUMP)
                gds = [pltpu.async_copy(y4_hbm.at[gbuf.at[j]], rows.at[j],
                                        semg) for j in range(16)]
                sds = []
                for j in range(16):
                    gds[j].wait()
                    sds.append(pltpu.async_copy(
                        rows.at[j], accum.at[dribuf.at[j]], sems, add=True))
                for d in sds:
                    d.wait()
                return 0
            lax.fori_loop(0, AGG_BLOCKS, block, 0)
            plsc.subcore_barrier()

            @pl.when(c == 0)
            def _():
                off = 0
                for sz in ZCH_H:
                    pltpu.sync_copy(accum.at[pl.ds(r0 + off, sz)],
                                    zbuf.at[pl.ds(0, sz)])
                    pltpu.sync_copy(
                        zbuf.at[pl.ds(0, sz)],
                        outs[si].at[pl.ds(lo + r0 + off, sz)])
                    off += sz

            @pl.when(c == 1)
            def _():
                off = 0
                for sz in ZCH_H:
                    pltpu.sync_copy(accum.at[pl.ds(r0 + off, sz)],
                                    zbuf.at[pl.ds(0, sz)])
                    pltpu.sync_copy(
                        zbuf.at[pl.ds(0, sz)],
                        outs[2 + si].at[pl.ds(lo + r0 + off, sz)])
                    off += sz
            plsc.subcore_barrier()

            lax.fori_loop(0, 1024, zinit, 0)


@functools.partial(
    pl.kernel,
    out_type=[
        jax.ShapeDtypeStruct((32 * G * H,), jnp.float32),
        jax.ShapeDtypeStruct((32 * G * H,), jnp.float32),
        jax.ShapeDtypeStruct((32 * G,), jnp.float32),
    ],
    mesh=_mesh,
    scratch_types=[
        pltpu.VMEM((512, H), jnp.float32),
        pltpu.VMEM((528,), jnp.int32),
        pltpu.VMEM((GP * H,), jnp.float32),
        pltpu.VMEM((GP * H,), jnp.float32),
        pltpu.VMEM((528,), jnp.float32),
    ],
    compiler_params=pltpu.CompilerParams(
        use_tc_tiling_on_sc=False, needs_layout_passes=False),
)
def _sc_pool(h_hbm, b_hbm, so_hbm, mo_hbm, co_hbm, hbuf, bbuf, sumf, maxf, cnt):
    c = lax.axis_index("c")
    s = lax.axis_index("s")
    wid = c * NS + s

    zero16 = jnp.zeros((16,), jnp.float32)
    ninf16 = jnp.full((16,), -jnp.inf, jnp.float32)

    def zinit(i, _):
        sumf[pl.ds(i * 16, 16)] = zero16
        maxf[pl.ds(i * 16, 16)] = ninf16
        return 0
    lax.fori_loop(0, GP * H // 16, zinit, 0)

    def cinit(i, _):
        cnt[pl.ds(i * 16, 16)] = zero16
        return 0
    lax.fori_loop(0, 33, cinit, 0)

    iota = lax.iota(jnp.int32, 16)
    lane0 = iota == 0
    ones16 = jnp.full((16,), 1.0, jnp.float32)

    row0 = wid * POOL_ROWS
    off = 0
    for sz in POOL_CH:
        pltpu.sync_copy(h_hbm.at[pl.ds(row0 + off, sz)],
                        hbuf.at[pl.ds(0, sz)])
        pltpu.sync_copy(b_hbm.at[pl.ds(row0 + off, sz)],
                        bbuf.at[pl.ds(0, sz)])

        def node(i, _):
            b = bbuf[pl.ds(i, 16)][0]
            base = b * H
            plsc.addupdate_scatter(cnt, [jnp.full((16,), b, jnp.int32)],
                                   ones16, mask=lane0)
            for k in range(4):
                idx = base + (16 * k + iota)
                row = hbuf[i, pl.ds(16 * k, 16)]
                plsc.addupdate_scatter(sumf, [idx], row)
                cur = plsc.load_gather(maxf, [idx])
                plsc.store_scatter(maxf, [idx], jnp.maximum(cur, row))
            return 0
        lax.fori_loop(0, sz, node, 0)
        off += sz

    pltpu.sync_copy(sumf.at[pl.ds(0, G * H)],
                    so_hbm.at[pl.ds(wid * G * H, G * H)])
    pltpu.sync_copy(maxf.at[pl.ds(0, G * H)],
                    mo_hbm.at[pl.ds(wid * G * H, G * H)])
    pltpu.sync_copy(cnt.at[pl.ds(0, G)], co_hbm.at[pl.ds(wid * G, G)])


_BLK = 2000
_NBLK = N // _BLK


def _tc_prep_body(d0, d1, x, w1, dinv_o, xw_o, y_o):
    deg = d0[...] + d1[...] + 1.0
    dinv = lax.rsqrt(deg)
    xw = jnp.dot(x[...], w1[...], preferred_element_type=jnp.float32)
    dinv_o[...] = dinv
    xw_o[...] = xw
    y_o[...] = xw * dinv


def _tc_prep(d0, d1, x, w1):
    return pl.pallas_call(
        _tc_prep_body,
        grid=(_NBLK,),
        in_specs=[
            pl.BlockSpec((_BLK, 1), lambda i: (i, 0)),
            pl.BlockSpec((_BLK, 1), lambda i: (i, 0)),
            pl.BlockSpec((_BLK, MOL), lambda i: (i, 0)),
            pl.BlockSpec((MOL, H), lambda i: (0, 0)),
        ],
        out_specs=[
            pl.BlockSpec((_BLK, 1), lambda i: (i, 0)),
            pl.BlockSpec((_BLK, H), lambda i: (i, 0)),
            pl.BlockSpec((_BLK, H), lambda i: (i, 0)),
        ],
        out_shape=[
            jax.ShapeDtypeStruct((N, 1), jnp.float32),
            jax.ShapeDtypeStruct((N, H), jnp.float32),
            jax.ShapeDtypeStruct((N, H), jnp.float32),
        ],
    )(d0, d1, x, w1)


def _tc_epi_body(a0, a1, a2, a3, xw, dinv, b, wn, xwn_o, yn_o):
    aggr = jnp.concatenate([a0[...], a1[...], a2[...], a3[...]], axis=1)
    dv = dinv[...]
    h = jnp.maximum(aggr * dv + xw[...] * (dv * dv) + b[...], 0.0)
    xwn = jnp.dot(h, wn[...], preferred_element_type=jnp.float32)
    xwn_o[...] = xwn
    yn_o[...] = xwn * dv


def _tc_epi(a, xw, dinv, b, wn):
    return pl.pallas_call(
        _tc_epi_body,
        grid=(_NBLK,),
        in_specs=[
            pl.BlockSpec((_BLK, 16), lambda i: (i, 0)),
            pl.BlockSpec((_BLK, 16), lambda i: (i, 0)),
            pl.BlockSpec((_BLK, 16), lambda i: (i, 0)),
            pl.BlockSpec((_BLK, 16), lambda i: (i, 0)),
            pl.BlockSpec((_BLK, H), lambda i: (i, 0)),
            pl.BlockSpec((_BLK, 1), lambda i: (i, 0)),
            pl.BlockSpec((1, H), lambda i: (0, 0)),
            pl.BlockSpec((H, H), lambda i: (0, 0)),
        ],
        out_specs=[
            pl.BlockSpec((_BLK, H), lambda i: (i, 0)),
            pl.BlockSpec((_BLK, H), lambda i: (i, 0)),
        ],
        out_shape=[
            jax.ShapeDtypeStruct((N, H), jnp.float32),
            jax.ShapeDtypeStruct((N, H), jnp.float32),
        ],
    )(a[0], a[1], a[2], a[3], xw, dinv, b, wn)


def _tc_epi3_body(a0, a1, a2, a3, xw, dinv, b, h_o):
    aggr = jnp.concatenate([a0[...], a1[...], a2[...], a3[...]], axis=1)
    dv = dinv[...]
    h_o[...] = jnp.maximum(aggr * dv + xw[...] * (dv * dv) + b[...], 0.0)


def _tc_epi3(a, xw, dinv, b):
    return pl.pallas_call(
        _tc_epi3_body,
        grid=(_NBLK,),
        in_specs=[
            pl.BlockSpec((_BLK, 16), lambda i: (i, 0)),
            pl.BlockSpec((_BLK, 16), lambda i: (i, 0)),
            pl.BlockSpec((_BLK, 16), lambda i: (i, 0)),
            pl.BlockSpec((_BLK, 16), lambda i: (i, 0)),
            pl.BlockSpec((_BLK, H), lambda i: (i, 0)),
            pl.BlockSpec((_BLK, 1), lambda i: (i, 0)),
            pl.BlockSpec((1, H), lambda i: (0, 0)),
        ],
        out_specs=pl.BlockSpec((_BLK, H), lambda i: (i, 0)),
        out_shape=jax.ShapeDtypeStruct((N, H), jnp.float32),
    )(a[0], a[1], a[2], a[3], xw, dinv, b)


def _tc_final_body(sums, maxs, cnts, wlin, blin, out_o):
    ssum = jnp.sum(sums[...], axis=0)
    smax = jnp.max(maxs[...], axis=0)
    scnt = jnp.sum(cnts[...], axis=0)
    mean = ssum / jnp.maximum(scnt, 1.0)
    pooled = jnp.concatenate([mean, smax], axis=1)
    out_o[...] = jnp.dot(pooled, wlin[...],
                         preferred_element_type=jnp.float32) + blin[...]


def _tc_final(sums, maxs, cnts, wlin, blin):
    return pl.pallas_call(
        _tc_final_body,
        grid=(8,),
        in_specs=[
            pl.BlockSpec((32, 64, H), lambda i: (0, i, 0)),
            pl.BlockSpec((32, 64, H), lambda i: (0, i, 0)),
            pl.BlockSpec((32, 64, 1), lambda i: (0, i, 0)),
            pl.BlockSpec((2 * H, H), lambda i: (0, 0)),
            pl.BlockSpec((1, H), lambda i: (0, 0)),
        ],
        out_specs=pl.BlockSpec((64, H), lambda i: (i, 0)),
        out_shape=jax.ShapeDtypeStruct((G, H), jnp.float32),
    )(sums, maxs, cnts, wlin, blin)


def kernel(x, edge_index, edge_attr, batch, W1, b1, W2, b2, W3, b3, Wlin, blin):
    src = edge_index[0]
    dst = edge_index[1]
    srcp = jnp.concatenate(
        [src, jnp.zeros((EPAD,), jnp.int32)]).reshape(EB, 128)
    dstp = jnp.concatenate(
        [dst, jnp.full((EPAD,), N, jnp.int32)]).reshape(EB, 128)

    degpart = _sc_deg(dstp)
    d0 = degpart[:N].reshape(N, 1)
    d1 = degpart[NR:NR + N].reshape(N, 1)
    dinv, xw1, y1 = _tc_prep(d0, d1, x, W1)

    a = _sc_aggr(srcp, dstp, y1.reshape(4 * N, 16))
    xw2, y2 = _tc_epi(a, xw1, dinv, b1.reshape(1, H), W2)
    a = _sc_aggr(srcp, dstp, y2.reshape(4 * N, 16))
    xw3, y3 = _tc_epi(a, xw2, dinv, b2.reshape(1, H), W3)
    a = _sc_aggr(srcp, dstp, y3.reshape(4 * N, 16))
    h3 = _tc_epi3(a, xw3, dinv, b3.reshape(1, H))

    hp = jnp.concatenate([h3, jnp.zeros((NP - N, H), jnp.float32)])
    bp = jnp.concatenate([batch, jnp.full((NP - N,), G, jnp.int32)])
    sums, maxs, cnts = _sc_pool(hp, bp)
    return _tc_final(sums.reshape(32, G, H), maxs.reshape(32, G, H),
                     cnts.reshape(32, G, 1), Wlin, blin.reshape(1, H))
